# baseline (speedup 1.0000x reference)
import functools

import jax
import jax.numpy as jnp
from jax import lax
from jax.experimental import pallas as pl
from jax.experimental.pallas import tpu as pltpu
from jax.experimental.pallas import tpu_sc as plsc

DIM = 64
EPS = 1e-5
NC = 2
NS = 16
NW = NC * NS
IG = 128
CHUNK = 256
GPC = CHUNK // IG


def _ln_chunk(rows, trans, tbuf, qbuf, abuf, cbuf, w_regs, b_regs, mask15,
              iota256):
    inv_d = 1.0 / DIM

    def pass1(r, carry):
        v0 = rows[r, pl.ds(0, 16)]
        v1 = rows[r, pl.ds(16, 16)]
        v2 = rows[r, pl.ds(32, 16)]
        v3 = rows[r, pl.ds(48, 16)]
        s = (v0 + v1) + (v2 + v3)
        q = (v0 * v0 + v1 * v1) + (v2 * v2 + v3 * v3)
        rsplat = jnp.full((16,), r, jnp.int32)
        plsc.store_scatter(tbuf, [rsplat], plsc.cumsum(s), mask=mask15)
        plsc.store_scatter(qbuf, [rsplat], plsc.cumsum(q), mask=mask15)
        return carry

    lax.fori_loop(0, CHUNK, pass1, 0, unroll=8)

    def pass2(g, carry):
        off = pl.multiple_of(g * 16, 16)
        t = tbuf[pl.ds(off, 16)]
        u = qbuf[pl.ds(off, 16)]
        mean = t * inv_d
        var = u * inv_d - mean * mean
        x = var + EPS
        i = plsc.bitcast(x, jnp.int32)
        i = jnp.int32(0x5F3759DF) - lax.shift_right_logical(i, 1)
        y = plsc.bitcast(i, jnp.float32)
        hx = x * 0.5
        y = y * (1.5 - hx * y * y)
        y = y * (1.5 - hx * y * y)
        abuf[pl.ds(off, 16)] = y
        cbuf[pl.ds(off, 16)] = -(mean * y)
        return carry

    lax.fori_loop(0, CHUNK // 16, pass2, 0, unroll=2)

    def pass3(r, carry):
        rsplat = jnp.full((16,), r, jnp.int32)
        a = plsc.load_gather(abuf, [rsplat])
        c = plsc.load_gather(cbuf, [rsplat])
        for k in range(4):
            n = rows[r, pl.ds(16 * k, 16)] * a + c
            plsc.store_scatter(trans, [iota256[k], rsplat],
                               n * w_regs[k] + b_regs[k])
        return carry

    lax.fori_loop(0, CHUNK, pass3, 0, unroll=8)


def _make_sc_kernel(b_dim, l_dim):
    b_per_w = b_dim // NW
    rows_per_w = b_per_w * l_dim
    half = b_per_w // CHUNK
    n_chunk = l_dim * half

    def body(table, idx1d, w_hbm, bias_hbm, out, idx_flat, idx_t,
             rows0, rows1, trans0, trans1,
             tbuf, qbuf, abuf, cbuf, wb_v,
             g0, g1, w0, w1):
        rows_bufs = (rows0, rows1)
        trans_bufs = (trans0, trans1)
        gsems = (g0, g1)
        wsems = (w0, w1)
        wid = lax.axis_index("s") * NC + lax.axis_index("c")
        b0 = wid * b_per_w
        pltpu.sync_copy(idx1d.at[pl.ds(wid * rows_per_w, rows_per_w)],
                        idx_flat)
        pltpu.sync_copy(w_hbm, wb_v.at[0])
        pltpu.sync_copy(bias_hbm, wb_v.at[1])
        w_regs = [wb_v[0, pl.ds(16 * k, 16)] for k in range(4)]
        b_regs = [wb_v[1, pl.ds(16 * k, 16)] for k in range(4)]
        mask15 = lax.iota(jnp.int32, 16) == 15
        iota16 = lax.iota(jnp.int32, 16)
        iota_l = iota16 * l_dim
        iota256 = [iota16 + 16 * k for k in range(4)]

        n_t16 = b_per_w // 16

        def tr(p, carry):
            l = p // n_t16
            b32 = p - l * n_t16
            src = b32 * (16 * l_dim) + iota_l + l
            v = plsc.load_gather(idx_flat, [src])
            dst = l * b_per_w + b32 * 16
            idx_t[dst // IG, pl.ds(dst % IG, 16)] = v
            return carry

        lax.fori_loop(0, l_dim * n_t16, tr, 0, unroll=4)

        def issue_gather(c, buf, sem):
            for j in range(GPC):
                pltpu.async_copy(table.at[idx_t.at[c * GPC + j]],
                                 buf.at[pl.ds(j * IG, IG)], sem)

        def wait_gather(buf, sem):
            pltpu.make_async_copy(table.at[pl.ds(0, CHUNK)], buf, sem).wait()

        def issue_wb(c, buf, sem):
            l = c // half
            h = c - l * half
            pltpu.async_copy(
                buf, out.at[l, :, pl.ds(b0 + h * CHUNK, CHUNK)], sem)

        def wait_wb(buf, sem):
            pltpu.make_async_copy(
                buf, out.at[0, :, pl.ds(0, CHUNK)], sem).wait()

        issue_gather(0, rows_bufs[0], gsems[0])

        def outer(i, carry):
            for k in range(2):
                c = i * 2 + k

                @pl.when(c + 1 < n_chunk)
                def _():
                    issue_gather(c + 1, rows_bufs[1 - k], gsems[1 - k])

                wait_gather(rows_bufs[k], gsems[k])

                @pl.when(c >= 2)
                def _():
                    wait_wb(trans_bufs[k], wsems[k])

                _ln_chunk(rows_bufs[k], trans_bufs[k], tbuf, qbuf, abuf, cbuf,
                          w_regs, b_regs, mask15, iota256)
                issue_wb(c, trans_bufs[k], wsems[k])
            return carry

        lax.fori_loop(0, n_chunk // 2, outer, 0)
        wait_wb(trans_bufs[0], wsems[0])
        wait_wb(trans_bufs[1], wsems[1])

    return pl.kernel(
        body,
        out_type=jax.ShapeDtypeStruct((l_dim, DIM, b_dim), jnp.float32),
        mesh=plsc.VectorSubcoreMesh(core_axis_name="c", subcore_axis_name="s"),
        compiler_params=pltpu.CompilerParams(
            needs_layout_passes=False, use_tc_tiling_on_sc=False),
        scratch_types=[
            pltpu.VMEM((rows_per_w,), jnp.int32),
            pltpu.VMEM((rows_per_w // IG, IG), jnp.int32),
            pltpu.VMEM((CHUNK, DIM), jnp.float32),
            pltpu.VMEM((CHUNK, DIM), jnp.float32),
            pltpu.VMEM((DIM, CHUNK), jnp.float32),
            pltpu.VMEM((DIM, CHUNK), jnp.float32),
            pltpu.VMEM((CHUNK,), jnp.float32),
            pltpu.VMEM((CHUNK,), jnp.float32),
            pltpu.VMEM((CHUNK,), jnp.float32),
            pltpu.VMEM((CHUNK,), jnp.float32),
            pltpu.VMEM((2, DIM), jnp.float32),
            pltpu.SemaphoreType.DMA,
            pltpu.SemaphoreType.DMA,
            pltpu.SemaphoreType.DMA,
            pltpu.SemaphoreType.DMA,
        ],
    )


def kernel(idx, table, ln_weight, ln_bias):
    b, l = idx.shape
    idx1d = idx.reshape(-1).astype(jnp.int32)
    out_phys = _make_sc_kernel(b, l)(table, idx1d, ln_weight, ln_bias)
    return jnp.transpose(out_phys, (2, 0, 1))


# speedup vs baseline: 2.0412x; 2.0412x over previous
import functools

import jax
import jax.numpy as jnp
from jax import lax
from jax.experimental import pallas as pl
from jax.experimental.pallas import tpu as pltpu
from jax.experimental.pallas import tpu_sc as plsc

DIM = 64
EPS = 1e-5
NC = 2
NS = 16
NW = NC * NS
IG = 128
CHUNK = 256
GPC = CHUNK // IG
NBUF = 4


def _ln_chunk(rows, tbuf, qbuf, abuf, cbuf, w_regs, b_regs, mask15):
    inv_d = 1.0 / DIM

    @plsc.parallel_loop(0, CHUNK, unroll=8)
    def pass1(r):
        v0 = rows[r, pl.ds(0, 16)]
        v1 = rows[r, pl.ds(16, 16)]
        v2 = rows[r, pl.ds(32, 16)]
        v3 = rows[r, pl.ds(48, 16)]
        s = (v0 + v1) + (v2 + v3)
        q = (v0 * v0 + v1 * v1) + (v2 * v2 + v3 * v3)
        rsplat = jnp.full((16,), r, jnp.int32)
        plsc.store_scatter(tbuf, [rsplat], plsc.cumsum(s), mask=mask15)
        plsc.store_scatter(qbuf, [rsplat], plsc.cumsum(q), mask=mask15)

    def pass2(g, carry):
        off = pl.multiple_of(g * 16, 16)
        t = tbuf[pl.ds(off, 16)]
        u = qbuf[pl.ds(off, 16)]
        mean = t * inv_d
        var = u * inv_d - mean * mean
        x = var + EPS
        i = plsc.bitcast(x, jnp.int32)
        i = jnp.int32(0x5F3759DF) - lax.shift_right_logical(i, 1)
        y = plsc.bitcast(i, jnp.float32)
        hx = x * 0.5
        y = y * (1.5 - hx * y * y)
        y = y * (1.5 - hx * y * y)
        abuf[pl.ds(off, 16)] = y
        cbuf[pl.ds(off, 16)] = -(mean * y)
        return carry

    lax.fori_loop(0, CHUNK // 16, pass2, 0, unroll=2)

    @plsc.parallel_loop(0, CHUNK, unroll=8)
    def pass3(r):
        rsplat = jnp.full((16,), r, jnp.int32)
        a = plsc.load_gather(abuf, [rsplat])
        c = plsc.load_gather(cbuf, [rsplat])
        for k in range(4):
            n = rows[r, pl.ds(16 * k, 16)] * a + c
            rows[r, pl.ds(16 * k, 16)] = n * w_regs[k] + b_regs[k]


def _make_sc_kernel(b_total):
    b_per_w = b_total // NW
    idx_groups = b_per_w // IG
    n_chunk = b_per_w // CHUNK
    n_outer = n_chunk // NBUF

    def body(table, idxg, w_hbm, bias_hbm, out, idx_v, r0, r1, r2, r3,
             tbuf, qbuf, abuf, cbuf, wb_v,
             g0, g1, g2, g3, w0, w1, w2, w3):
        bufs = (r0, r1, r2, r3)
        gsems = (g0, g1, g2, g3)
        wsems = (w0, w1, w2, w3)
        wid = lax.axis_index("s") * NC + lax.axis_index("c")
        row0 = wid * b_per_w
        pltpu.sync_copy(idxg.at[pl.ds(wid * idx_groups, idx_groups)], idx_v)
        pltpu.sync_copy(w_hbm, wb_v.at[0])
        pltpu.sync_copy(bias_hbm, wb_v.at[1])
        w_regs = [wb_v[0, pl.ds(16 * k, 16)] for k in range(4)]
        b_regs = [wb_v[1, pl.ds(16 * k, 16)] for k in range(4)]
        mask15 = lax.iota(jnp.int32, 16) == 15

        def issue_gather(c, buf, sem):
            for j in range(GPC):
                pltpu.async_copy(table.at[idx_v.at[c * GPC + j]],
                                 buf.at[pl.ds(j * IG, IG)], sem)

        def wait_gather(buf, sem):
            pltpu.make_async_copy(out.at[pl.ds(0, CHUNK)], buf, sem).wait()

        def issue_wb(c, buf, sem):
            pltpu.async_copy(buf, out.at[pl.ds(row0 + c * CHUNK, CHUNK)], sem)

        def wait_wb(buf, sem):
            pltpu.make_async_copy(buf, out.at[pl.ds(0, CHUNK)], sem).wait()

        issue_gather(0, bufs[0], gsems[0])
        issue_gather(1, bufs[1], gsems[1])

        def outer(i, carry):
            for k in range(NBUF):
                c = i * NBUF + k
                b = k
                wait_gather(bufs[b], gsems[b])
                _ln_chunk(bufs[b], tbuf, qbuf, abuf, cbuf,
                          w_regs, b_regs, mask15)
                issue_wb(c, bufs[b], wsems[b])
                b2 = (k + 2) % NBUF

                @pl.when(jnp.logical_and(c + 2 < n_chunk, c >= 2))
                def _():
                    wait_wb(bufs[b2], wsems[b2])

                @pl.when(c + 2 < n_chunk)
                def _():
                    issue_gather(c + 2, bufs[b2], gsems[b2])

            return carry

        lax.fori_loop(0, n_outer, outer, 0)
        wait_wb(bufs[(n_chunk - 2) % NBUF], wsems[(n_chunk - 2) % NBUF])
        wait_wb(bufs[(n_chunk - 1) % NBUF], wsems[(n_chunk - 1) % NBUF])

    return pl.kernel(
        body,
        out_type=jax.ShapeDtypeStruct((b_total, DIM), jnp.float32),
        mesh=plsc.VectorSubcoreMesh(core_axis_name="c", subcore_axis_name="s"),
        compiler_params=pltpu.CompilerParams(
            needs_layout_passes=False, use_tc_tiling_on_sc=False),
        scratch_types=[
            pltpu.VMEM((idx_groups, IG), jnp.int32),
            pltpu.VMEM((CHUNK, DIM), jnp.float32),
            pltpu.VMEM((CHUNK, DIM), jnp.float32),
            pltpu.VMEM((CHUNK, DIM), jnp.float32),
            pltpu.VMEM((CHUNK, DIM), jnp.float32),
            pltpu.VMEM((CHUNK,), jnp.float32),
            pltpu.VMEM((CHUNK,), jnp.float32),
            pltpu.VMEM((CHUNK,), jnp.float32),
            pltpu.VMEM((CHUNK,), jnp.float32),
            pltpu.VMEM((2, DIM), jnp.float32),
            pltpu.SemaphoreType.DMA,
            pltpu.SemaphoreType.DMA,
            pltpu.SemaphoreType.DMA,
            pltpu.SemaphoreType.DMA,
            pltpu.SemaphoreType.DMA,
            pltpu.SemaphoreType.DMA,
            pltpu.SemaphoreType.DMA,
            pltpu.SemaphoreType.DMA,
        ],
    )


def kernel(idx, table, ln_weight, ln_bias):
    b, l = idx.shape
    b_total = b * l
    idxg = idx.reshape(b_total // IG, IG).astype(jnp.int32)
    out = _make_sc_kernel(b_total)(table, idxg, ln_weight, ln_bias)
    return out.reshape(b, l, DIM)
